# baseline (speedup 1.0000x reference)
import functools

import jax
import jax.numpy as jnp
from jax import lax
from jax.experimental import pallas as pl
from jax.experimental.pallas import tpu as pltpu
from jax.experimental.pallas import tpu_sc as plsc

NUM_SUBCAT = 100000
EMBED_DIM = 100
PROJ_DIM = 400
BATCH = 16384

_CHUNK = 128


def _make_gather(batch, embed_dim):
  info = plsc.get_sparse_core_info()
  nw = info.num_cores * info.num_subcores
  b_per_w = batch // nw
  n_chunks = b_per_w // _CHUNK
  mesh = plsc.VectorSubcoreMesh(core_axis_name="c", subcore_axis_name="s")

  @functools.partial(
      pl.kernel,
      mesh=mesh,
      out_type=jax.ShapeDtypeStruct((batch, embed_dim), jnp.float32),
      scratch_types=[
          pltpu.VMEM((b_per_w,), jnp.int32),
          pltpu.VMEM((b_per_w, embed_dim), jnp.float32),
          pltpu.SemaphoreType.DMA,
      ],
  )
  def gather_kernel(table_hbm, idx_hbm, out_hbm, idx_v, rows_v, sem):
    wid = lax.axis_index("s") * info.num_cores + lax.axis_index("c")
    base = wid * b_per_w
    pltpu.sync_copy(idx_hbm.at[pl.ds(base, b_per_w)], idx_v)
    copies = []
    for j in range(n_chunks):
      copies.append(
          pltpu.async_copy(
              table_hbm.at[idx_v.at[pl.ds(j * _CHUNK, _CHUNK)]],
              rows_v.at[pl.ds(j * _CHUNK, _CHUNK)],
              sem,
          ))
    for c in copies:
      c.wait()
    pltpu.sync_copy(rows_v, out_hbm.at[pl.ds(base, b_per_w)])

  return gather_kernel


_gather = _make_gather(BATCH, EMBED_DIM)


def _proj_body(x_ref, w_ref, b_ref, o_ref):
  acc = jnp.dot(x_ref[...], w_ref[...], preferred_element_type=jnp.float32)
  o_ref[...] = jnp.maximum(acc + b_ref[...], 0.0)


def _projection(emb, W, b2d, block_rows=1024):
  batch = emb.shape[0]
  grid = (batch // block_rows,)
  return pl.pallas_call(
      _proj_body,
      grid=grid,
      in_specs=[
          pl.BlockSpec((block_rows, EMBED_DIM), lambda i: (i, 0)),
          pl.BlockSpec((EMBED_DIM, PROJ_DIM), lambda i: (0, 0)),
          pl.BlockSpec((1, PROJ_DIM), lambda i: (0, 0)),
      ],
      out_specs=pl.BlockSpec((block_rows, PROJ_DIM), lambda i: (i, 0)),
      out_shape=jax.ShapeDtypeStruct((batch, PROJ_DIM), jnp.float32),
  )(emb, W, b2d)


@jax.jit
def kernel(inputs, table, W, b):
  idx = inputs.reshape(-1).astype(jnp.int32)
  emb = jnp.take(table, idx, axis=0)
  return _projection(emb, W, b.reshape(1, PROJ_DIM))
